# Initial kernel scaffold; baseline (speedup 1.0000x reference)
import jax
import jax.numpy as jnp
from jax.experimental import pallas as pl


def kernel(x, pos, batch, W1, b1, g1, be1, W2, b2, g2, be2, W3, b3, g3, be3, Wc1, bc1, gc1, bec1, Wc2, bc2, gc2, bec2):
    raise NotImplementedError("write your pallas kernel here")


# speedup vs baseline: 4.8058x; 4.8058x over previous
import functools

import jax
import jax.numpy as jnp
from jax import lax
from jax.experimental import pallas as pl
from jax.experimental.pallas import tpu as pltpu
from jax.experimental.pallas import tpu_sc as plsc

G, P = 16, 256
EPS = 1e-5
KPAD = 16


def _knn_body(k, f_ref, idx_ref):
    f = f_ref[0]
    sq = jnp.sum(f * f, axis=1, keepdims=True)
    d2 = sq + jnp.reshape(sq, (1, P)) - 2.0 * lax.dot_general(
        f, f, (((1,), (1,)), ((), ())))
    row = lax.broadcasted_iota(jnp.int32, (P, P), 0)
    cols = lax.broadcasted_iota(jnp.int32, (P, P), 1)
    d2 = d2 + jnp.where(row == cols, jnp.float32(1e9), jnp.float32(0.0))
    outs = []
    for _ in range(k):
        m = jnp.min(d2, axis=1, keepdims=True)
        amin = jnp.min(jnp.where(d2 <= m, cols, P), axis=1, keepdims=True)
        outs.append(amin)
        d2 = jnp.where(cols == amin, jnp.float32(3e38), d2)
    outs += [jnp.zeros((P, 1), jnp.int32)] * (KPAD - k)
    idx_ref[0] = jnp.concatenate(outs, axis=1)


def _knn(f, k):
    C = f.shape[-1]
    return pl.pallas_call(
        functools.partial(_knn_body, k),
        grid=(G,),
        in_specs=[pl.BlockSpec((1, P, C), lambda g: (g, 0, 0))],
        out_specs=pl.BlockSpec((1, P, KPAD), lambda g: (g, 0, 0)),
        out_shape=jax.ShapeDtypeStruct((G, P, KPAD), jnp.int32),
    )(f.reshape(G, P, C))


def _gather_dif(x, idx, K):
    C = x.shape[1]
    split_cols = C >= 256
    if split_cols:
        RH, CH = P, C // 2
    else:
        RH, CH = P // 2, C
    CS = CH // 16
    CHN = 8
    NCHUNK = RH // CHN
    mesh = plsc.VectorSubcoreMesh(core_axis_name="c", subcore_axis_name="s")

    @functools.partial(
        pl.kernel,
        out_type=jax.ShapeDtypeStruct((G * P * K, C), jnp.float32),
        mesh=mesh,
        scratch_types=[
            pltpu.VMEM((P, KPAD), jnp.int32),
            pltpu.VMEM((P, CH), jnp.float32),
            pltpu.VMEM((CHN * K, CH), jnp.float32),
        ],
    )
    def gather_kernel(x_hbm, idx_hbm, dif_hbm, idx_v, x_v, d_v):
        wid = lax.axis_index("s") * 2 + lax.axis_index("c")
        g = wid // 2
        hf = wid % 2
        if split_cols:
            lo, co = 0, hf * CH
        else:
            lo, co = hf * RH, 0
        pltpu.sync_copy(idx_hbm.at[g], idx_v)
        pltpu.sync_copy(x_hbm.at[pl.ds(g * P, P), pl.ds(co, CH)], x_v)

        def chunk_body(ch, carry):
            c0 = ch * CHN

            def node_body(i, carry2):
                ib = lo + c0 + i
                iv = idx_v[ib, :]
                xi = [x_v[ib, pl.ds(cs * 16, 16)] for cs in range(CS)]
                for j in range(K):
                    r = iv[j]
                    for cs in range(CS):
                        d_v[i * K + j, pl.ds(cs * 16, 16)] = (
                            x_v[r, pl.ds(cs * 16, 16)] - xi[cs])
                return carry2

            lax.fori_loop(0, CHN, node_body, 0)
            pltpu.sync_copy(
                d_v,
                dif_hbm.at[pl.ds((g * P + lo + c0) * K, CHN * K),
                           pl.ds(co, CH)])
            return carry

        lax.fori_loop(0, NCHUNK, chunk_body, 0)

    return gather_kernel(x, idx)


def _edge_reduce_body(K, dif_ref, an_ref, Wb_ref, s_ref, s2p_ref):
    Cout = Wb_ref.shape[1]
    t = jnp.dot(dif_ref[...], Wb_ref[...], preferred_element_type=jnp.float32)
    e = jnp.maximum(t.reshape(P, K, Cout) + an_ref[...][:, None, :], 0.0)
    s_ref[...] = jnp.sum(e, axis=1)
    s2p_ref[...] = jnp.sum(e * e, axis=(0, 1)).reshape(1, 1, Cout)


def _edge_reduce(dif, anode, Wb, K):
    Cin = dif.shape[1]
    Cout = Wb.shape[1]
    return pl.pallas_call(
        functools.partial(_edge_reduce_body, K),
        grid=(G,),
        in_specs=[
            pl.BlockSpec((P * K, Cin), lambda g: (g, 0)),
            pl.BlockSpec((P, Cout), lambda g: (g, 0)),
            pl.BlockSpec((Cin, Cout), lambda g: (0, 0)),
        ],
        out_specs=[
            pl.BlockSpec((P, Cout), lambda g: (g, 0)),
            pl.BlockSpec((1, 1, Cout), lambda g: (g, 0, 0)),
        ],
        out_shape=[
            jax.ShapeDtypeStruct((G * P, Cout), jnp.float32),
            jax.ShapeDtypeStruct((G, 1, Cout), jnp.float32),
        ],
    )(dif, anode, Wb)


def _edge_reduce2(dif, anode, Wb, K):
    s, s2p = _edge_reduce(dif, anode, Wb, K)
    return s, s2p.reshape(G, Wb.shape[1])


def _bnproj_body(kprev, s_ref, s2p_ref, g_ref, be_ref, W_ref, bias_ref,
                 h_ref, an_ref):
    s = s_ref[...]
    n = jnp.float32(G * P * kprev)
    m = jnp.sum(s, axis=0) / n
    v = jnp.sum(s2p_ref[...], axis=0) / n - m * m
    h = (s * jnp.float32(1.0 / kprev) - m) * lax.rsqrt(v + EPS) * g_ref[...] \
        + be_ref[...]
    h_ref[...] = h
    an_ref[...] = (
        jnp.dot(h, W_ref[...], preferred_element_type=jnp.float32)
        + bias_ref[...]
    )


def _bnproj(s, s2p, g, be, kprev, Wa, bias):
    Cp = s.shape[1]
    return pl.pallas_call(
        functools.partial(_bnproj_body, kprev),
        out_shape=[
            jax.ShapeDtypeStruct((G * P, Cp), jnp.float32),
            jax.ShapeDtypeStruct((G * P, Wa.shape[1]), jnp.float32),
        ],
    )(s, s2p, g, be, Wa, bias)


def _anode1_body(x_ref, W_ref, b_ref, o_ref):
    o_ref[...] = (
        jnp.dot(x_ref[...], W_ref[...], preferred_element_type=jnp.float32)
        + b_ref[...]
    )


def _anode1(x, Wa, b):
    return pl.pallas_call(
        _anode1_body,
        out_shape=jax.ShapeDtypeStruct((G * P, Wa.shape[1]), jnp.float32),
    )(x, Wa, b)


def _head_body(s_ref, s2p_ref, g3_ref, be3_ref, Wc1_ref, bc1_ref, gc1_ref,
               bec1_ref, Wc2_ref, bc2_ref, gc2_ref, bec2_ref, o_ref):
    s = s_ref[...]
    n = jnp.float32(G * P * 16)
    m = jnp.sum(s, axis=0) / n
    v = jnp.sum(s2p_ref[...], axis=0) / n - m * m
    h = (s * jnp.float32(1.0 / 16.0) - m) * lax.rsqrt(v + EPS) * g3_ref[...] \
        + be3_ref[...]
    pooled = jnp.mean(h.reshape(G, P, h.shape[-1]), axis=1)
    c = jnp.maximum(
        jnp.dot(pooled, Wc1_ref[...], preferred_element_type=jnp.float32)
        + bc1_ref[...], 0.0)
    m1 = jnp.mean(c, axis=0)
    v1 = jnp.mean(jnp.square(c - m1), axis=0)
    c = (c - m1) * lax.rsqrt(v1 + EPS) * gc1_ref[...] + bec1_ref[...]
    c2 = jnp.maximum(
        jnp.dot(c, Wc2_ref[...], preferred_element_type=jnp.float32)
        + bc2_ref[...], 0.0)
    m2 = jnp.mean(c2, axis=0)
    v2 = jnp.mean(jnp.square(c2 - m2), axis=0)
    c2 = (c2 - m2) * lax.rsqrt(v2 + EPS) * gc2_ref[...] + bec2_ref[...]
    o_ref[...] = jax.nn.sigmoid(c2[:, 0])


def _head(s, s2p, g3, be3, Wc1, bc1, gc1, bec1, Wc2, bc2, gc2, bec2):
    return pl.pallas_call(
        _head_body,
        out_shape=jax.ShapeDtypeStruct((G,), jnp.float32),
    )(s, s2p, g3, be3, Wc1, bc1, gc1, bec1, Wc2, bc2, gc2, bec2)


def kernel(x, pos, batch, W1, b1, g1, be1, W2, b2, g2, be2, W3, b3, g3, be3,
           Wc1, bc1, gc1, bec1, Wc2, bc2, gc2, bec2):
    del batch

    x48 = jnp.pad(x, ((0, 0), (0, 13)))
    Wb1 = jnp.pad(W1[35:], ((0, 13), (0, 0)))
    idx = _knn(pos, 12)
    an = _anode1(x, W1[:35], b1)
    dif = _gather_dif(x48, idx, 12)
    s, s2p = _edge_reduce2(dif, an, Wb1, 12)

    h, an = _bnproj(s, s2p, g1, be1, 12, W2[:128], b2)
    idx = _knn(h, 14)
    dif = _gather_dif(h, idx, 14)
    s, s2p = _edge_reduce2(dif, an, W2[128:], 14)

    h, an = _bnproj(s, s2p, g2, be2, 14, W3[:256], b3)
    idx = _knn(h, 16)
    dif = _gather_dif(h, idx, 16)
    s, s2p = _edge_reduce2(dif, an, W3[256:], 16)

    return _head(s, s2p, g3, be3, Wc1, bc1, gc1, bec1, Wc2, bc2, gc2, bec2)


# speedup vs baseline: 5.5149x; 1.1475x over previous
import functools

import jax
import jax.numpy as jnp
from jax import lax
from jax.experimental import pallas as pl
from jax.experimental.pallas import tpu as pltpu
from jax.experimental.pallas import tpu_sc as plsc

G, P = 16, 256
EPS = 1e-5
KPAD = 16


def _knn_body(k, f_ref, idx_ref):
    f = f_ref[0]
    gbase = pl.program_id(0) * P
    sq = jnp.sum(f * f, axis=1, keepdims=True)
    d2 = sq + jnp.reshape(sq, (1, P)) - 2.0 * lax.dot_general(
        f, f, (((1,), (1,)), ((), ())))
    row = lax.broadcasted_iota(jnp.int32, (P, P), 0)
    cols = lax.broadcasted_iota(jnp.int32, (P, P), 1)
    d2 = d2 + jnp.where(row == cols, jnp.float32(1e9), jnp.float32(0.0))
    outs = []
    for _ in range(k):
        m = jnp.min(d2, axis=1, keepdims=True)
        amin = jnp.min(jnp.where(d2 <= m, cols, P), axis=1, keepdims=True)
        outs.append(amin + gbase)
        d2 = jnp.where(cols == amin, jnp.float32(3e38), d2)
    outs += [jnp.full((P, 1), gbase, jnp.int32)] * (KPAD - k)
    idx_ref[0] = jnp.concatenate(outs, axis=1)


def _knn(f, k):
    C = f.shape[-1]
    return pl.pallas_call(
        functools.partial(_knn_body, k),
        grid=(G,),
        in_specs=[pl.BlockSpec((1, P, C), lambda g: (g, 0, 0))],
        out_specs=pl.BlockSpec((1, P, KPAD), lambda g: (g, 0, 0)),
        out_shape=jax.ShapeDtypeStruct((G, P, KPAD), jnp.int32),
    )(f.reshape(G, P, C))


def _gather_rows(x, idxflat):
    C = x.shape[1]
    EH = P * KPAD // 2
    CHN = 128
    NCH = EH // CHN
    mesh = plsc.VectorSubcoreMesh(core_axis_name="c", subcore_axis_name="s")

    @functools.partial(
        pl.kernel,
        out_type=jax.ShapeDtypeStruct((G * P * KPAD, C), jnp.float32),
        mesh=mesh,
        scratch_types=[
            pltpu.VMEM((EH,), jnp.int32),
            pltpu.VMEM((CHN, C), jnp.float32),
            pltpu.VMEM((CHN, C), jnp.float32),
            pltpu.SemaphoreType.DMA,
            pltpu.SemaphoreType.DMA,
        ],
    )
    def gather_kernel(x_hbm, idx_hbm, xj_hbm, idx_v, r0, r1, s0, s1):
        wid = lax.axis_index("s") * 2 + lax.axis_index("c")
        g = wid // 2
        hf = wid % 2
        base = g * P * KPAD + hf * EH
        pltpu.sync_copy(idx_hbm.at[g, pl.ds(hf * EH, EH)], idx_v)
        bufs = ((r0, s0), (r1, s1))
        pend = [None, None]
        pend[0] = pltpu.async_copy(
            x_hbm.at[idx_v.at[pl.ds(0, CHN)]], r0, s0)
        for c in range(NCH):
            if c + 1 < NCH:
                nb, ns = bufs[(c + 1) % 2]
                pend[(c + 1) % 2] = pltpu.async_copy(
                    x_hbm.at[idx_v.at[pl.ds((c + 1) * CHN, CHN)]], nb, ns)
            pend[c % 2].wait()
            pltpu.sync_copy(bufs[c % 2][0],
                            xj_hbm.at[pl.ds(base + c * CHN, CHN), :])

    return gather_kernel(x, idxflat)


def _edge_reduce_body(K, xj_ref, xi_ref, an_ref, Wb_ref, s_ref, s2p_ref):
    Cin = xi_ref.shape[1]
    Cout = Wb_ref.shape[1]
    dif = xj_ref[...].reshape(P, KPAD, Cin) - xi_ref[...][:, None, :]
    t = jnp.dot(dif.reshape(P * KPAD, Cin), Wb_ref[...],
                preferred_element_type=jnp.float32)
    e = jnp.maximum(t.reshape(P, KPAD, Cout) + an_ref[...][:, None, :], 0.0)
    e = e[:, :K, :]
    s_ref[...] = jnp.sum(e, axis=1)
    s2p_ref[...] = jnp.sum(e * e, axis=(0, 1)).reshape(1, 1, Cout)


def _edge_reduce(xj, xi, anode, Wb, K):
    Cin = xi.shape[1]
    Cout = Wb.shape[1]
    return pl.pallas_call(
        functools.partial(_edge_reduce_body, K),
        grid=(G,),
        in_specs=[
            pl.BlockSpec((P * KPAD, Cin), lambda g: (g, 0)),
            pl.BlockSpec((P, Cin), lambda g: (g, 0)),
            pl.BlockSpec((P, Cout), lambda g: (g, 0)),
            pl.BlockSpec((Cin, Cout), lambda g: (0, 0)),
        ],
        out_specs=[
            pl.BlockSpec((P, Cout), lambda g: (g, 0)),
            pl.BlockSpec((1, 1, Cout), lambda g: (g, 0, 0)),
        ],
        out_shape=[
            jax.ShapeDtypeStruct((G * P, Cout), jnp.float32),
            jax.ShapeDtypeStruct((G, 1, Cout), jnp.float32),
        ],
    )(xj, xi, anode, Wb)


def _edge_reduce2(xj, xi, anode, Wb, K):
    s, s2p = _edge_reduce(xj, xi, anode, Wb, K)
    return s, s2p.reshape(G, Wb.shape[1])


def _bnproj_body(kprev, s_ref, s2p_ref, g_ref, be_ref, W_ref, bias_ref,
                 h_ref, an_ref):
    s = s_ref[...]
    n = jnp.float32(G * P * kprev)
    m = jnp.sum(s, axis=0) / n
    v = jnp.sum(s2p_ref[...], axis=0) / n - m * m
    h = (s * jnp.float32(1.0 / kprev) - m) * lax.rsqrt(v + EPS) * g_ref[...] \
        + be_ref[...]
    h_ref[...] = h
    an_ref[...] = (
        jnp.dot(h, W_ref[...], preferred_element_type=jnp.float32)
        + bias_ref[...]
    )


def _bnproj(s, s2p, g, be, kprev, Wa, bias):
    Cp = s.shape[1]
    return pl.pallas_call(
        functools.partial(_bnproj_body, kprev),
        out_shape=[
            jax.ShapeDtypeStruct((G * P, Cp), jnp.float32),
            jax.ShapeDtypeStruct((G * P, Wa.shape[1]), jnp.float32),
        ],
    )(s, s2p, g, be, Wa, bias)


def _anode1_body(x_ref, W_ref, b_ref, o_ref):
    o_ref[...] = (
        jnp.dot(x_ref[...], W_ref[...], preferred_element_type=jnp.float32)
        + b_ref[...]
    )


def _anode1(x, Wa, b):
    return pl.pallas_call(
        _anode1_body,
        out_shape=jax.ShapeDtypeStruct((G * P, Wa.shape[1]), jnp.float32),
    )(x, Wa, b)


def _head_body(s_ref, s2p_ref, g3_ref, be3_ref, Wc1_ref, bc1_ref, gc1_ref,
               bec1_ref, Wc2_ref, bc2_ref, gc2_ref, bec2_ref, o_ref):
    s = s_ref[...]
    n = jnp.float32(G * P * 16)
    m = jnp.sum(s, axis=0) / n
    v = jnp.sum(s2p_ref[...], axis=0) / n - m * m
    h = (s * jnp.float32(1.0 / 16.0) - m) * lax.rsqrt(v + EPS) * g3_ref[...] \
        + be3_ref[...]
    pooled = jnp.mean(h.reshape(G, P, h.shape[-1]), axis=1)
    c = jnp.maximum(
        jnp.dot(pooled, Wc1_ref[...], preferred_element_type=jnp.float32)
        + bc1_ref[...], 0.0)
    m1 = jnp.mean(c, axis=0)
    v1 = jnp.mean(jnp.square(c - m1), axis=0)
    c = (c - m1) * lax.rsqrt(v1 + EPS) * gc1_ref[...] + bec1_ref[...]
    c2 = jnp.maximum(
        jnp.dot(c, Wc2_ref[...], preferred_element_type=jnp.float32)
        + bc2_ref[...], 0.0)
    m2 = jnp.mean(c2, axis=0)
    v2 = jnp.mean(jnp.square(c2 - m2), axis=0)
    c2 = (c2 - m2) * lax.rsqrt(v2 + EPS) * gc2_ref[...] + bec2_ref[...]
    o_ref[...] = jax.nn.sigmoid(c2[:, 0])


def _head(s, s2p, g3, be3, Wc1, bc1, gc1, bec1, Wc2, bc2, gc2, bec2):
    return pl.pallas_call(
        _head_body,
        out_shape=jax.ShapeDtypeStruct((G,), jnp.float32),
    )(s, s2p, g3, be3, Wc1, bc1, gc1, bec1, Wc2, bc2, gc2, bec2)


def kernel(x, pos, batch, W1, b1, g1, be1, W2, b2, g2, be2, W3, b3, g3, be3,
           Wc1, bc1, gc1, bec1, Wc2, bc2, gc2, bec2):
    del batch

    x128 = jnp.pad(x, ((0, 0), (0, 93)))
    Wb1 = jnp.pad(W1[35:], ((0, 93), (0, 0)))
    idx = _knn(pos, 12).reshape(G, P * KPAD)
    an = _anode1(x, W1[:35], b1)
    xj = _gather_rows(x128, idx)
    s, s2p = _edge_reduce2(xj, x128, an, Wb1, 12)

    h, an = _bnproj(s, s2p, g1, be1, 12, W2[:128], b2)
    idx = _knn(h, 14).reshape(G, P * KPAD)
    xj = _gather_rows(h, idx)
    s, s2p = _edge_reduce2(xj, h, an, W2[128:], 14)

    h, an = _bnproj(s, s2p, g2, be2, 14, W3[:256], b3)
    idx = _knn(h, 16).reshape(G, P * KPAD)
    xj = _gather_rows(h, idx)
    s, s2p = _edge_reduce2(xj, h, an, W3[256:], 16)

    return _head(s, s2p, g3, be3, Wc1, bc1, gc1, bec1, Wc2, bc2, gc2, bec2)


# speedup vs baseline: 6.1977x; 1.1238x over previous
import functools

import jax
import jax.numpy as jnp
from jax import lax
from jax.experimental import pallas as pl
from jax.experimental.pallas import tpu as pltpu
from jax.experimental.pallas import tpu_sc as plsc

G, P = 16, 256
EPS = 1e-5
KPAD = 16


def _knn_body(k, f_ref, idx_ref):
    f = f_ref[0]
    gbase = pl.program_id(0) * P
    sq = jnp.sum(f * f, axis=1, keepdims=True)
    d2 = sq + jnp.reshape(sq, (1, P)) - 2.0 * lax.dot_general(
        f, f, (((1,), (1,)), ((), ())))
    row = lax.broadcasted_iota(jnp.int32, (P, P), 0)
    cols = lax.broadcasted_iota(jnp.int32, (P, P), 1)
    d2 = d2 + jnp.where(row == cols, jnp.float32(1e9), jnp.float32(0.0))
    outs = []
    for _ in range(k):
        m = jnp.min(d2, axis=1, keepdims=True)
        amin = jnp.min(jnp.where(d2 <= m, cols, P), axis=1, keepdims=True)
        outs.append(amin + gbase)
        d2 = jnp.where(cols == amin, jnp.float32(3e38), d2)
    idx_ref[0] = jnp.concatenate(outs, axis=1)


def _knn(f, k):
    C = f.shape[-1]
    return pl.pallas_call(
        functools.partial(_knn_body, k),
        grid=(G,),
        in_specs=[pl.BlockSpec((1, P, C), lambda g: (g, 0, 0))],
        out_specs=pl.BlockSpec((1, P, k), lambda g: (g, 0, 0)),
        out_shape=jax.ShapeDtypeStruct((G, P, k), jnp.int32),
    )(f.reshape(G, P, C))


def _gather_rows(x, idxflat, K):
    C = x.shape[1]
    EH = P * K // 2
    CHN = 128
    NCH = EH // CHN
    NBUF = min(3 if C > 128 else 4, NCH)
    mesh = plsc.VectorSubcoreMesh(core_axis_name="c", subcore_axis_name="s")

    @functools.partial(
        pl.kernel,
        out_type=jax.ShapeDtypeStruct((G * P * K, C), jnp.float32),
        mesh=mesh,
        scratch_types=[pltpu.VMEM((EH,), jnp.int32)]
        + [pltpu.VMEM((CHN, C), jnp.float32)] * NBUF
        + [pltpu.SemaphoreType.DMA] * NBUF,
    )
    def gather_kernel(x_hbm, idx_hbm, xj_hbm, idx_v, *scr):
        rows = scr[:NBUF]
        sems = scr[NBUF:]
        wid = lax.axis_index("s") * 2 + lax.axis_index("c")
        g = wid // 2
        hf = wid % 2
        base = g * P * K + hf * EH
        pltpu.sync_copy(idx_hbm.at[g, pl.ds(hf * EH, EH)], idx_v)

        def fire(c):
            return pltpu.async_copy(
                x_hbm.at[idx_v.at[pl.ds(c * CHN, CHN)]],
                rows[c % NBUF], sems[c % NBUF])

        pend = [fire(c) for c in range(NBUF)]
        for c in range(NCH):
            pend[c % NBUF].wait()
            pltpu.sync_copy(rows[c % NBUF],
                            xj_hbm.at[pl.ds(base + c * CHN, CHN), :])
            if c + NBUF < NCH:
                pend[c % NBUF] = fire(c + NBUF)

    return gather_kernel(x, idxflat)


def _edge_reduce_body(K, xj_ref, xi_ref, an_ref, Wb_ref, s_ref, s2p_ref):
    Cin = xi_ref.shape[1]
    Cout = Wb_ref.shape[1]
    dif = xj_ref[...].reshape(P, K, Cin) - xi_ref[...][:, None, :]
    t = jnp.dot(dif.reshape(P * K, Cin), Wb_ref[...],
                preferred_element_type=jnp.float32)
    e = jnp.maximum(t.reshape(P, K, Cout) + an_ref[...][:, None, :], 0.0)
    s_ref[...] = jnp.sum(e, axis=1)
    s2p_ref[...] = jnp.sum(e * e, axis=(0, 1)).reshape(1, 1, Cout)


def _edge_reduce(xj, xi, anode, Wb, K):
    Cin = xi.shape[1]
    Cout = Wb.shape[1]
    return pl.pallas_call(
        functools.partial(_edge_reduce_body, K),
        grid=(G,),
        in_specs=[
            pl.BlockSpec((P * K, Cin), lambda g: (g, 0)),
            pl.BlockSpec((P, Cin), lambda g: (g, 0)),
            pl.BlockSpec((P, Cout), lambda g: (g, 0)),
            pl.BlockSpec((Cin, Cout), lambda g: (0, 0)),
        ],
        out_specs=[
            pl.BlockSpec((P, Cout), lambda g: (g, 0)),
            pl.BlockSpec((1, 1, Cout), lambda g: (g, 0, 0)),
        ],
        out_shape=[
            jax.ShapeDtypeStruct((G * P, Cout), jnp.float32),
            jax.ShapeDtypeStruct((G, 1, Cout), jnp.float32),
        ],
    )(xj, xi, anode, Wb)


def _edge_reduce2(xj, xi, anode, Wb, K):
    s, s2p = _edge_reduce(xj, xi, anode, Wb, K)
    return s, s2p.reshape(G, Wb.shape[1])


def _bnproj_body(kprev, s_ref, s2p_ref, g_ref, be_ref, W_ref, bias_ref,
                 h_ref, an_ref):
    s = s_ref[...]
    n = jnp.float32(G * P * kprev)
    m = jnp.sum(s, axis=0) / n
    v = jnp.sum(s2p_ref[...], axis=0) / n - m * m
    h = (s * jnp.float32(1.0 / kprev) - m) * lax.rsqrt(v + EPS) * g_ref[...] \
        + be_ref[...]
    h_ref[...] = h
    an_ref[...] = (
        jnp.dot(h, W_ref[...], preferred_element_type=jnp.float32)
        + bias_ref[...]
    )


def _bnproj(s, s2p, g, be, kprev, Wa, bias):
    Cp = s.shape[1]
    return pl.pallas_call(
        functools.partial(_bnproj_body, kprev),
        out_shape=[
            jax.ShapeDtypeStruct((G * P, Cp), jnp.float32),
            jax.ShapeDtypeStruct((G * P, Wa.shape[1]), jnp.float32),
        ],
    )(s, s2p, g, be, Wa, bias)


def _anode1_body(x_ref, W_ref, b_ref, o_ref):
    o_ref[...] = (
        jnp.dot(x_ref[...], W_ref[...], preferred_element_type=jnp.float32)
        + b_ref[...]
    )


def _anode1(x, Wa, b):
    return pl.pallas_call(
        _anode1_body,
        out_shape=jax.ShapeDtypeStruct((G * P, Wa.shape[1]), jnp.float32),
    )(x, Wa, b)


def _head_body(s_ref, s2p_ref, g3_ref, be3_ref, Wc1_ref, bc1_ref, gc1_ref,
               bec1_ref, Wc2_ref, bc2_ref, gc2_ref, bec2_ref, o_ref):
    s = s_ref[...]
    n = jnp.float32(G * P * 16)
    m = jnp.sum(s, axis=0) / n
    v = jnp.sum(s2p_ref[...], axis=0) / n - m * m
    h = (s * jnp.float32(1.0 / 16.0) - m) * lax.rsqrt(v + EPS) * g3_ref[...] \
        + be3_ref[...]
    pooled = jnp.mean(h.reshape(G, P, h.shape[-1]), axis=1)
    c = jnp.maximum(
        jnp.dot(pooled, Wc1_ref[...], preferred_element_type=jnp.float32)
        + bc1_ref[...], 0.0)
    m1 = jnp.mean(c, axis=0)
    v1 = jnp.mean(jnp.square(c - m1), axis=0)
    c = (c - m1) * lax.rsqrt(v1 + EPS) * gc1_ref[...] + bec1_ref[...]
    c2 = jnp.maximum(
        jnp.dot(c, Wc2_ref[...], preferred_element_type=jnp.float32)
        + bc2_ref[...], 0.0)
    m2 = jnp.mean(c2, axis=0)
    v2 = jnp.mean(jnp.square(c2 - m2), axis=0)
    c2 = (c2 - m2) * lax.rsqrt(v2 + EPS) * gc2_ref[...] + bec2_ref[...]
    o_ref[...] = jax.nn.sigmoid(c2[:, 0])


def _head(s, s2p, g3, be3, Wc1, bc1, gc1, bec1, Wc2, bc2, gc2, bec2):
    return pl.pallas_call(
        _head_body,
        out_shape=jax.ShapeDtypeStruct((G,), jnp.float32),
    )(s, s2p, g3, be3, Wc1, bc1, gc1, bec1, Wc2, bc2, gc2, bec2)


def kernel(x, pos, batch, W1, b1, g1, be1, W2, b2, g2, be2, W3, b3, g3, be3,
           Wc1, bc1, gc1, bec1, Wc2, bc2, gc2, bec2):
    del batch

    x128 = jnp.pad(x, ((0, 0), (0, 93)))
    Wb1 = jnp.pad(W1[35:], ((0, 93), (0, 0)))
    idx = _knn(pos, 12).reshape(G, P * 12)
    an = _anode1(x, W1[:35], b1)
    xj = _gather_rows(x128, idx, 12)
    s, s2p = _edge_reduce2(xj, x128, an, Wb1, 12)

    h, an = _bnproj(s, s2p, g1, be1, 12, W2[:128], b2)
    idx = _knn(h, 14).reshape(G, P * 14)
    xj = _gather_rows(h, idx, 14)
    s, s2p = _edge_reduce2(xj, h, an, W2[128:], 14)

    h, an = _bnproj(s, s2p, g2, be2, 14, W3[:256], b3)
    idx = _knn(h, 16).reshape(G, P * 16)
    xj = _gather_rows(h, idx, 16)
    s, s2p = _edge_reduce2(xj, h, an, W3[256:], 16)

    return _head(s, s2p, g3, be3, Wc1, bc1, gc1, bec1, Wc2, bc2, gc2, bec2)
